# Initial kernel scaffold; baseline (speedup 1.0000x reference)
import jax
import jax.numpy as jnp
from jax.experimental import pallas as pl


def kernel(x, embed):
    raise NotImplementedError("write your pallas kernel here")


# speedup vs baseline: 1.1397x; 1.1397x over previous
import jax
import jax.numpy as jnp
from jax import lax
from jax.experimental import pallas as pl
from jax.experimental.pallas import tpu as pltpu

_EMB_DIM = 256
_NUM_EMB = 1024
_COMMIT = 0.25


def _vq_body(x_ref, emb_ref, quant_ref, idx_ref, loss_ref):
    b = pl.program_id(0)
    xb = x_ref[0]
    emb = emb_ref[...]

    e2 = jnp.sum(emb * emb, axis=0)
    xe = lax.dot_general(
        emb, xb,
        dimension_numbers=(((0,), (0,)), ((), ())),
        preferred_element_type=jnp.float32,
        precision=lax.Precision.DEFAULT,
    )
    scores = e2[:, None] - 2.0 * xe

    minval = jnp.min(scores, axis=0)
    iota_j = lax.broadcasted_iota(jnp.int32, (_NUM_EMB, _NUM_EMB), 0)
    masked = jnp.where(scores == minval[None, :], iota_j, _NUM_EMB)
    idx = jnp.min(masked, axis=0).astype(jnp.int32)
    idx_ref[0, 0, :] = idx

    onehot = (iota_j == idx[None, :]).astype(jnp.float32)
    quant = lax.dot_general(
        emb, onehot,
        dimension_numbers=(((1,), (0,)), ((), ())),
        preferred_element_type=jnp.float32,
        precision=lax.Precision.HIGHEST,
    )
    quant_ref[0] = quant

    part = jnp.sum(xb * xb) + jnp.sum(minval)

    @pl.when(b == 0)
    def _():
        loss_ref[0, 0] = 0.0

    loss_ref[0, 0] += part


def kernel(x, embed):
    B, C, H, W = x.shape
    HW = H * W
    x3 = x.reshape(B, C, HW)

    quant, idx3, loss_sum = pl.pallas_call(
        _vq_body,
        grid=(B,),
        in_specs=[
            pl.BlockSpec((1, C, HW), lambda i: (i, 0, 0)),
            pl.BlockSpec((_EMB_DIM, _NUM_EMB), lambda i: (0, 0)),
        ],
        out_specs=[
            pl.BlockSpec((1, C, HW), lambda i: (i, 0, 0)),
            pl.BlockSpec((1, 1, HW), lambda i: (i, 0, 0)),
            pl.BlockSpec((1, 1), lambda i: (0, 0), memory_space=pltpu.SMEM),
        ],
        out_shape=[
            jax.ShapeDtypeStruct((B, C, HW), jnp.float32),
            jax.ShapeDtypeStruct((B, 1, HW), jnp.int32),
            jax.ShapeDtypeStruct((1, 1), jnp.float32),
        ],
    )(x3, embed)

    quantize = quant.reshape(B, C, H, W)
    embed_idx = idx3.reshape(B, H, W)
    loss = loss_sum[0, 0] * (_COMMIT / (B * HW * C))
    return quantize, loss, embed_idx


# speedup vs baseline: 1.6587x; 1.4554x over previous
import jax
import jax.numpy as jnp
from jax import lax
from jax.experimental import pallas as pl
from jax.experimental.pallas import tpu as pltpu

_EMB_DIM = 256
_NUM_EMB = 1024
_COMMIT = 0.25


def _vq_body(x_ref, emb_ref, hi_ref, lo_ref, quant_ref, idx_ref, loss_ref):
    b = pl.program_id(0)
    xb = x_ref[0]
    emb = emb_ref[...]

    e2 = jnp.sum(emb * emb, axis=0)
    xe = lax.dot_general(
        emb, xb,
        dimension_numbers=(((0,), (0,)), ((), ())),
        preferred_element_type=jnp.float32,
        precision=lax.Precision.DEFAULT,
    )
    scores = e2[:, None] - 2.0 * xe

    minval = jnp.min(scores, axis=0)
    idx = jnp.argmin(scores, axis=0).astype(jnp.int32)
    idx_ref[0, 0, :] = idx

    iota_j = lax.broadcasted_iota(jnp.int32, (_NUM_EMB, _NUM_EMB), 0)
    onehot = (iota_j == idx[None, :]).astype(jnp.bfloat16)
    quant = lax.dot_general(
        hi_ref[...], onehot,
        dimension_numbers=(((1,), (0,)), ((), ())),
        preferred_element_type=jnp.float32,
    ) + lax.dot_general(
        lo_ref[...], onehot,
        dimension_numbers=(((1,), (0,)), ((), ())),
        preferred_element_type=jnp.float32,
    )
    quant_ref[0] = quant

    part = jnp.sum(xb * xb) + jnp.sum(minval)

    @pl.when(b == 0)
    def _():
        loss_ref[0, 0] = 0.0

    loss_ref[0, 0] += part


def kernel(x, embed):
    B, C, H, W = x.shape
    HW = H * W
    x3 = x.reshape(B, C, HW)
    emb_hi = embed.astype(jnp.bfloat16)
    emb_lo = (embed - emb_hi.astype(jnp.float32)).astype(jnp.bfloat16)

    quant, idx3, loss_sum = pl.pallas_call(
        _vq_body,
        grid=(B,),
        in_specs=[
            pl.BlockSpec((1, C, HW), lambda i: (i, 0, 0)),
            pl.BlockSpec((_EMB_DIM, _NUM_EMB), lambda i: (0, 0)),
            pl.BlockSpec((_EMB_DIM, _NUM_EMB), lambda i: (0, 0)),
            pl.BlockSpec((_EMB_DIM, _NUM_EMB), lambda i: (0, 0)),
        ],
        out_specs=[
            pl.BlockSpec((1, C, HW), lambda i: (i, 0, 0)),
            pl.BlockSpec((1, 1, HW), lambda i: (i, 0, 0)),
            pl.BlockSpec((1, 1), lambda i: (0, 0), memory_space=pltpu.SMEM),
        ],
        out_shape=[
            jax.ShapeDtypeStruct((B, C, HW), jnp.float32),
            jax.ShapeDtypeStruct((B, 1, HW), jnp.int32),
            jax.ShapeDtypeStruct((1, 1), jnp.float32),
        ],
    )(x3, embed, emb_hi, emb_lo)

    quantize = quant.reshape(B, C, H, W)
    embed_idx = idx3.reshape(B, H, W)
    loss = loss_sum[0, 0] * (_COMMIT / (B * HW * C))
    return quantize, loss, embed_idx
